# baseline (speedup 1.0000x reference)
import jax
import jax.numpy as jnp
from jax.experimental import pallas as pl

_HB = 8


def _exchange_kernel(x1_ref, x2_ref, x3_ref, o1_ref, o2_ref, o3_ref):
    m = jax.lax.broadcasted_iota(jnp.int32, x1_ref.shape, 3) & 3
    m0 = m == 0
    m2 = m == 2
    a = x1_ref[...]
    b = x2_ref[...]
    c = x3_ref[...]
    o1_ref[...] = jnp.where(m0, c, jnp.where(m2, b, a))
    o2_ref[...] = jnp.where(m0, c, jnp.where(m2, a, b))
    o3_ref[...] = jnp.where(m0, b, jnp.where(m2, a, c))


def kernel(x1, x2, x3):
    b, ch, h, w = x1.shape
    t = lambda x: x.transpose(0, 2, 3, 1)
    spec = pl.BlockSpec((1, _HB, w, ch), lambda i, j: (i, j, 0, 0))
    y1, y2, y3 = pl.pallas_call(
        _exchange_kernel,
        grid=(b, h // _HB),
        in_specs=[spec, spec, spec],
        out_specs=[spec, spec, spec],
        out_shape=[jax.ShapeDtypeStruct((b, h, w, ch), x1.dtype)] * 3,
    )(t(x1), t(x2), t(x3))
    u = lambda y: y.transpose(0, 3, 1, 2)
    return (u(y1), u(y2), u(y3))


# speedup vs baseline: 1.1028x; 1.1028x over previous
import jax
import jax.numpy as jnp
from jax.experimental import pallas as pl

_HB = 32


def _exchange_kernel(x1_ref, x2_ref, x3_ref, o1_ref, o2_ref, o3_ref):
    m = jax.lax.broadcasted_iota(jnp.int32, x1_ref.shape, 3) & 3
    m0 = m == 0
    m2 = m == 2
    a = x1_ref[...]
    b = x2_ref[...]
    c = x3_ref[...]
    o1_ref[...] = jnp.where(m0, c, jnp.where(m2, b, a))
    o2_ref[...] = jnp.where(m0, c, jnp.where(m2, a, b))
    o3_ref[...] = jnp.where(m0, b, jnp.where(m2, a, c))


def kernel(x1, x2, x3):
    b, ch, h, w = x1.shape
    t = lambda x: x.transpose(0, 2, 3, 1)
    spec = pl.BlockSpec((1, _HB, w, ch), lambda i, j: (i, j, 0, 0))
    y1, y2, y3 = pl.pallas_call(
        _exchange_kernel,
        grid=(b, h // _HB),
        in_specs=[spec, spec, spec],
        out_specs=[spec, spec, spec],
        out_shape=[jax.ShapeDtypeStruct((b, h, w, ch), x1.dtype)] * 3,
    )(t(x1), t(x2), t(x3))
    u = lambda y: y.transpose(0, 3, 1, 2)
    return (u(y1), u(y2), u(y3))
